# baseline (speedup 1.0000x reference)
import functools

import jax
import jax.numpy as jnp
from jax import lax
from jax.experimental import pallas as pl
from jax.experimental.pallas import tpu as pltpu
from jax.experimental.pallas import tpu_sc as plsc

F_N, L_N, E_N, V_N = 10000, 20000, 40000, 30000
EMB = 64
K = 6
C = 256
NSUB = 16
RB = 40
TCB = 1000


@functools.cache
def _make_seg_sum(N_src, N_dst, NQ):
    H = N_dst // 2
    A = H + RB
    NCH = NQ // NSUB
    NZ = A // RB
    NW = H // RB
    mesh = plsc.VectorSubcoreMesh(core_axis_name="c", subcore_axis_name="s")

    @functools.partial(
        pl.kernel,
        out_type=jax.ShapeDtypeStruct((2, A, EMB), jnp.float32),
        mesh=mesh,
        compiler_params=pltpu.CompilerParams(use_tc_tiling_on_sc=False),
        scratch_types=[
            pltpu.VMEM_SHARED((A, EMB), jnp.float32),
            pltpu.VMEM((C,), jnp.int32), pltpu.VMEM((C,), jnp.int32),
            pltpu.VMEM((C,), jnp.int32), pltpu.VMEM((C,), jnp.int32),
            pltpu.VMEM((C, EMB), jnp.float32), pltpu.VMEM((C, EMB), jnp.float32),
            pltpu.VMEM((RB, EMB), jnp.float32),
            pltpu.SemaphoreType.DMA, pltpu.SemaphoreType.DMA,
        ],
    )
    def seg_sum(x_hbm, src_hbm, dst_hbm, out_hbm,
                acc, ib0, ib1, db0, db1, rb0, rb1, zb, sem0, sem1):
        c = lax.axis_index("c")
        s = lax.axis_index("s")
        ibs, dbs, rbs, sems = (ib0, ib1), (db0, db1), (rb0, rb1), (sem0, sem1)

        z16 = jnp.zeros((16,), jnp.float32)
        for r in range(RB):
            for q in range(EMB // 16):
                zb[r, pl.ds(q * 16, 16)] = z16

        def zbody(j, carry):
            cid = j * NSUB + s
            @pl.when(cid < NZ)
            def _():
                pltpu.sync_copy(zb, acc.at[pl.ds(cid * RB, RB)])
            return carry
        lax.fori_loop(0, (NZ + NSUB - 1) // NSUB, zbody, 0)
        plsc.subcore_barrier()

        q0 = s * NCH
        pltpu.sync_copy(src_hbm.at[q0], ib0)
        pltpu.sync_copy(dst_hbm.at[c, q0], db0)
        pltpu.async_copy(x_hbm.at[ib0], rb0, sem0)

        def body(j, carry):
            for b in range(2):
                jj = j * 2 + b
                cur, nxt = b, 1 - b
                @pl.when(jj + 1 < NCH)
                def _():
                    qn = s * NCH + jj + 1
                    pltpu.sync_copy(src_hbm.at[qn], ibs[nxt])
                    pltpu.sync_copy(dst_hbm.at[c, qn], dbs[nxt])
                    pltpu.async_copy(x_hbm.at[ibs[nxt]], rbs[nxt], sems[nxt])
                pltpu.make_async_copy(x_hbm.at[ibs[cur]], rbs[cur],
                                      sems[cur]).wait()
                pltpu.sync_copy(rbs[cur], acc.at[dbs[cur]], add=True)
            return carry
        lax.fori_loop(0, NCH // 2, body, 0)
        plsc.subcore_barrier()

        def wbody(j, carry):
            cid = j * NSUB + s
            @pl.when(cid < NW)
            def _():
                pltpu.sync_copy(acc.at[pl.ds(cid * RB, RB)], zb)
                pltpu.sync_copy(zb, out_hbm.at[c, pl.ds(cid * RB, RB)])
            return carry
        lax.fori_loop(0, (NW + NSUB - 1) // NSUB, wbody, 0)

    return seg_sum


def _prep_dir(src_idx, dst_idx, N_dst):
    n = src_idx.shape[0]
    H = N_dst // 2
    n_pad = -(-n // (NSUB * C * 2)) * (NSUB * C * 2)
    pad = n_pad - n
    src_p = jnp.pad(src_idx, (0, pad))
    dst_p = jnp.pad(dst_idx, (0, pad), constant_values=-1)
    own0 = (dst_p >= 0) & (dst_p < H)
    own1 = dst_p >= H
    d0 = jnp.where(own0, dst_p, H)
    d1 = jnp.where(own1, dst_p - H, H)
    NQ = n_pad // C
    return (src_p.reshape(NQ, C),
            jnp.stack([d0, d1]).reshape(2, NQ, C).astype(jnp.int32), NQ)


def _seg_sum(x, src2d, dst3d, NQ, N_dst):
    return _make_seg_sum(x.shape[0], N_dst, NQ)(x, src2d, dst3d)


def _embed_body(x_ref, w_ref, b_ref, o_ref):
    o_ref[...] = jnp.maximum(
        jnp.dot(x_ref[...], w_ref[...], preferred_element_type=jnp.float32)
        + b_ref[...], 0.0)


@functools.cache
def _make_embed(N, S):
    return pl.pallas_call(
        _embed_body,
        grid=(N // TCB,),
        in_specs=[pl.BlockSpec((TCB, S), lambda i: (i, 0)),
                  pl.BlockSpec((S, EMB), lambda i: (0, 0)),
                  pl.BlockSpec((1, EMB), lambda i: (0, 0))],
        out_specs=pl.BlockSpec((TCB, EMB), lambda i: (i, 0)),
        out_shape=jax.ShapeDtypeStruct((N, EMB), jnp.float32),
    )


def _embed(x, w, b):
    return _make_embed(x.shape[0], x.shape[1])(x, w, b.reshape(1, EMB))


def _stage_body(d_ref, s_ref, w_ref, o_ref):
    o_ref[...] = jnp.maximum(
        d_ref[...] + jnp.dot(s_ref[0], w_ref[...],
                             preferred_element_type=jnp.float32), 0.0)


@functools.cache
def _make_stage(N, A):
    HB = (N // 2) // TCB
    return pl.pallas_call(
        _stage_body,
        grid=(N // TCB,),
        in_specs=[pl.BlockSpec((TCB, EMB), lambda i: (i, 0)),
                  pl.BlockSpec((1, TCB, EMB), lambda i: (i // HB, i % HB, 0)),
                  pl.BlockSpec((EMB, EMB), lambda i: (0, 0))],
        out_specs=pl.BlockSpec((TCB, EMB), lambda i: (i, 0)),
        out_shape=jax.ShapeDtypeStruct((N, EMB), jnp.float32),
    )


def _stage(dst, x, src2d, dst3d, NQ, W):
    N_dst = dst.shape[0]
    s2 = _seg_sum(x, src2d, dst3d, NQ, N_dst)
    return _make_stage(N_dst, s2.shape[1])(dst, s2, W)


def kernel(left_faces, left_loops, left_edges, left_verts,
           right_faces, right_loops, right_edges, right_verts,
           left_face_to_loop, left_loop_to_edge, left_edge_to_vertex,
           left_face_to_face, right_face_to_loop, right_loop_to_edge,
           right_edge_to_vertex, right_face_to_face,
           Wf, bf, Wl, bl, We, be, Wv, bv,
           W_ve, W_el, W_lf, W_ff, W_fl, W_le, W_ev):
    def side(faces, loops, edges, verts, f2l, l2e, e2v, f2f):
        f = _embed(faces, Wf, bf)
        l = _embed(loops, Wl, bl)
        e = _embed(edges, We, be)
        v = _embed(verts, Wv, bv)
        up_ve = _prep_dir(e2v[1], e2v[0], E_N)
        up_el = _prep_dir(l2e[1], l2e[0], L_N)
        up_lf = _prep_dir(f2l[1], f2l[0], F_N)
        up_ff = _prep_dir(f2f[1], f2f[0], F_N)
        dn_fl = _prep_dir(f2l[0], f2l[1], L_N)
        dn_le = _prep_dir(l2e[0], l2e[1], E_N)
        dn_ev = _prep_dir(e2v[0], e2v[1], V_N)
        for _ in range(K):
            e = _stage(e, v, *up_ve, W_ve)
            l = _stage(l, e, *up_el, W_el)
            f = _stage(f, l, *up_lf, W_lf)
            f = _stage(f, f, *up_ff, W_ff)
            l = _stage(l, f, *dn_fl, W_fl)
            e = _stage(e, l, *dn_le, W_le)
            v = _stage(v, e, *dn_ev, W_ev)
        return f, e, v

    out_l = side(left_faces, left_loops, left_edges, left_verts,
                 left_face_to_loop, left_loop_to_edge, left_edge_to_vertex,
                 left_face_to_face)
    out_r = side(right_faces, right_loops, right_edges, right_verts,
                 right_face_to_loop, right_loop_to_edge, right_edge_to_vertex,
                 right_face_to_face)
    return (out_l, out_r)


# speedup vs baseline: 1.5826x; 1.5826x over previous
import functools

import jax
import jax.numpy as jnp
from jax import lax
from jax.experimental import pallas as pl
from jax.experimental.pallas import tpu as pltpu
from jax.experimental.pallas import tpu_sc as plsc

F_N, L_N, E_N, V_N = 10000, 20000, 40000, 30000
EMB = 64
K = 6
C = 128
NSUB = 16
RB = 40
TCB = 1000


@functools.cache
def _make_seg_sum(N_src, N_dst, NQ):
    H = N_dst // 2
    A = H + RB
    NCH = NQ // NSUB
    NZ = A // RB
    NW = H // RB
    mesh = plsc.VectorSubcoreMesh(core_axis_name="c", subcore_axis_name="s")

    @functools.partial(
        pl.kernel,
        out_type=jax.ShapeDtypeStruct((2, A, EMB), jnp.float32),
        mesh=mesh,
        compiler_params=pltpu.CompilerParams(use_tc_tiling_on_sc=False),
        scratch_types=[
            pltpu.VMEM_SHARED((A, EMB), jnp.float32),
            pltpu.VMEM((2, C), jnp.int32), pltpu.VMEM((2, C), jnp.int32),
            pltpu.VMEM((C, EMB), jnp.float32), pltpu.VMEM((C, EMB), jnp.float32),
            pltpu.VMEM((RB, EMB), jnp.float32),
            pltpu.SemaphoreType.DMA, pltpu.SemaphoreType.DMA,
        ],
    )
    def seg_sum(x_hbm, idx_hbm, out_hbm,
                acc, pb0, pb1, rb0, rb1, zb, sem0, sem1):
        c = lax.axis_index("c")
        s = lax.axis_index("s")
        pbs, rbs, sems = (pb0, pb1), (rb0, rb1), (sem0, sem1)

        z16 = jnp.zeros((16,), jnp.float32)
        for r in range(RB):
            for q in range(EMB // 16):
                zb[r, pl.ds(q * 16, 16)] = z16

        def zbody(j, carry):
            cid = j * NSUB + s
            @pl.when(cid < NZ)
            def _():
                pltpu.sync_copy(zb, acc.at[pl.ds(cid * RB, RB)])
            return carry
        lax.fori_loop(0, (NZ + NSUB - 1) // NSUB, zbody, 0)
        plsc.subcore_barrier()

        q0 = s * NCH
        pltpu.sync_copy(idx_hbm.at[c, q0], pb0)
        pltpu.async_copy(x_hbm.at[pb0.at[0]], rb0, sem0)

        def body(j, carry):
            for b in range(2):
                jj = j * 2 + b
                cur, nxt = b, 1 - b
                @pl.when(jj + 1 < NCH)
                def _():
                    qn = s * NCH + jj + 1
                    pltpu.sync_copy(idx_hbm.at[c, qn], pbs[nxt])
                    pltpu.async_copy(x_hbm.at[pbs[nxt].at[0]],
                                     rbs[nxt], sems[nxt])
                pltpu.make_async_copy(x_hbm.at[pbs[cur].at[0]], rbs[cur],
                                      sems[cur]).wait()
                pltpu.sync_copy(rbs[cur], acc.at[pbs[cur].at[1]], add=True)
            return carry
        lax.fori_loop(0, NCH // 2, body, 0)
        plsc.subcore_barrier()

        def wbody(j, carry):
            cid = j * NSUB + s
            @pl.when(cid < NW)
            def _():
                pltpu.sync_copy(acc.at[pl.ds(cid * RB, RB)], zb)
                pltpu.sync_copy(zb, out_hbm.at[c, pl.ds(cid * RB, RB)])
            return carry
        lax.fori_loop(0, (NW + NSUB - 1) // NSUB, wbody, 0)

    return seg_sum


def _prep_dir(src_idx, dst_idx, N_dst):
    n = src_idx.shape[0]
    H = N_dst // 2
    n_pad = -(-n // (NSUB * C * 2)) * (NSUB * C * 2)
    pad = n_pad - n
    src_p = jnp.pad(src_idx, (0, pad))
    dst_p = jnp.pad(dst_idx, (0, pad), constant_values=-1)
    own0 = (dst_p >= 0) & (dst_p < H)
    own1 = dst_p >= H
    d0 = jnp.where(own0, dst_p, H)
    d1 = jnp.where(own1, dst_p - H, H)
    NQ = n_pad // C
    s2 = src_p.reshape(NQ, 1, C)
    pk = jnp.stack([
        jnp.concatenate([s2, d0.reshape(NQ, 1, C)], axis=1),
        jnp.concatenate([s2, d1.reshape(NQ, 1, C)], axis=1)]).astype(jnp.int32)
    return (pk, NQ)


def _seg_sum(x, pk, NQ, N_dst):
    return _make_seg_sum(x.shape[0], N_dst, NQ)(x, pk)


def _embed_body(x_ref, w_ref, b_ref, o_ref):
    o_ref[...] = jnp.maximum(
        jnp.dot(x_ref[...], w_ref[...], preferred_element_type=jnp.float32)
        + b_ref[...], 0.0)


@functools.cache
def _make_embed(N, S):
    return pl.pallas_call(
        _embed_body,
        grid=(N // TCB,),
        in_specs=[pl.BlockSpec((TCB, S), lambda i: (i, 0)),
                  pl.BlockSpec((S, EMB), lambda i: (0, 0)),
                  pl.BlockSpec((1, EMB), lambda i: (0, 0))],
        out_specs=pl.BlockSpec((TCB, EMB), lambda i: (i, 0)),
        out_shape=jax.ShapeDtypeStruct((N, EMB), jnp.float32),
    )


def _embed(x, w, b):
    return _make_embed(x.shape[0], x.shape[1])(x, w, b.reshape(1, EMB))


def _stage_body(d_ref, s_ref, w_ref, o_ref):
    o_ref[...] = jnp.maximum(
        d_ref[...] + jnp.dot(s_ref[0], w_ref[...],
                             preferred_element_type=jnp.float32), 0.0)


@functools.cache
def _make_stage(N, A):
    HB = (N // 2) // TCB
    return pl.pallas_call(
        _stage_body,
        grid=(N // TCB,),
        in_specs=[pl.BlockSpec((TCB, EMB), lambda i: (i, 0)),
                  pl.BlockSpec((1, TCB, EMB), lambda i: (i // HB, i % HB, 0)),
                  pl.BlockSpec((EMB, EMB), lambda i: (0, 0))],
        out_specs=pl.BlockSpec((TCB, EMB), lambda i: (i, 0)),
        out_shape=jax.ShapeDtypeStruct((N, EMB), jnp.float32),
    )


def _stage(dst, x, pk, NQ, W):
    N_dst = dst.shape[0]
    s2 = _seg_sum(x, pk, NQ, N_dst)
    return _make_stage(N_dst, s2.shape[1])(dst, s2, W)


def kernel(left_faces, left_loops, left_edges, left_verts,
           right_faces, right_loops, right_edges, right_verts,
           left_face_to_loop, left_loop_to_edge, left_edge_to_vertex,
           left_face_to_face, right_face_to_loop, right_loop_to_edge,
           right_edge_to_vertex, right_face_to_face,
           Wf, bf, Wl, bl, We, be, Wv, bv,
           W_ve, W_el, W_lf, W_ff, W_fl, W_le, W_ev):
    def side(faces, loops, edges, verts, f2l, l2e, e2v, f2f):
        f = _embed(faces, Wf, bf)
        l = _embed(loops, Wl, bl)
        e = _embed(edges, We, be)
        v = _embed(verts, Wv, bv)
        up_ve = _prep_dir(e2v[1], e2v[0], E_N)
        up_el = _prep_dir(l2e[1], l2e[0], L_N)
        up_lf = _prep_dir(f2l[1], f2l[0], F_N)
        up_ff = _prep_dir(f2f[1], f2f[0], F_N)
        dn_fl = _prep_dir(f2l[0], f2l[1], L_N)
        dn_le = _prep_dir(l2e[0], l2e[1], E_N)
        dn_ev = _prep_dir(e2v[0], e2v[1], V_N)
        for _ in range(K):
            e = _stage(e, v, *up_ve, W_ve)
            l = _stage(l, e, *up_el, W_el)
            f = _stage(f, l, *up_lf, W_lf)
            f = _stage(f, f, *up_ff, W_ff)
            l = _stage(l, f, *dn_fl, W_fl)
            e = _stage(e, l, *dn_le, W_le)
            v = _stage(v, e, *dn_ev, W_ev)
        return f, e, v

    out_l = side(left_faces, left_loops, left_edges, left_verts,
                 left_face_to_loop, left_loop_to_edge, left_edge_to_vertex,
                 left_face_to_face)
    out_r = side(right_faces, right_loops, right_edges, right_verts,
                 right_face_to_loop, right_loop_to_edge, right_edge_to_vertex,
                 right_face_to_face)
    return (out_l, out_r)
